# baseline (speedup 1.0000x reference)
import functools

import jax
import jax.numpy as jnp
from jax import lax
from jax.experimental import pallas as pl
from jax.experimental.pallas import tpu as pltpu
from jax.experimental.pallas import tpu_sc as plsc

B, S, D = 4, 8192, 8
V = 64
N_TOK = B * S
NW = 32
TPW = N_TOK // NW
MAGIC = jnp.float32(12582912.0)


@functools.partial(
    pl.kernel,
    mesh=plsc.VectorSubcoreMesh(core_axis_name="c", subcore_axis_name="s"),
    out_type=(
        jax.ShapeDtypeStruct((N_TOK * V,), jnp.float32),
        jax.ShapeDtypeStruct((N_TOK * D,), jnp.float32),
    ),
    compiler_params=pltpu.CompilerParams(needs_layout_passes=False),
    scratch_types=[
        pltpu.VMEM((TPW * D,), jnp.float32),
        pltpu.VMEM((TPW * V,), jnp.float32),
        pltpu.SemaphoreType.DMA,
        pltpu.SemaphoreType.DMA,
    ],
)
def _sc_fake_logits(x_hbm, out_hbm, hid_hbm, x_v, o_v, in_sem, out_sem):
    cid = lax.axis_index("c")
    sid = lax.axis_index("s")
    wid = sid * 2 + cid
    lane = lax.iota(jnp.int32, 16)

    in_cp = pltpu.async_copy(
        x_hbm.at[pl.ds(wid * (TPW * D), TPW * D)], x_v, in_sem
    )

    zeros = jnp.zeros((16,), jnp.float32)

    def zbody(i, carry):
        o_v[pl.ds(i * 16, 16)] = zeros
        return carry

    lax.fori_loop(0, TPW * V // 16, zbody, 0, unroll=8)
    in_cp.wait()

    hid_cp = pltpu.async_copy(
        x_v, hid_hbm.at[pl.ds(wid * (TPW * D), TPW * D)], in_sem
    )

    def gbody(g, carry):
        xv = plsc.load_gather(x_v, [g * (16 * D) + lane * D])
        r = (xv + MAGIC) - MAGIC
        r = jnp.maximum(r, jnp.float32(0.0))
        idx = r.astype(jnp.int32) & (V - 1)
        val = idx.astype(jnp.float32) / jnp.float32(10.0)
        plsc.store_scatter(o_v, [g * (16 * V) + lane * V + idx], val)
        return carry

    lax.fori_loop(0, TPW // 16, gbody, 0, unroll=4)

    pltpu.async_copy(o_v, out_hbm.at[pl.ds(wid * (TPW * V), TPW * V)], out_sem).wait()
    hid_cp.wait()


def kernel(inputs_embeds):
    logits_flat, hidden_flat = _sc_fake_logits(inputs_embeds.reshape(-1))
    return logits_flat.reshape(B, S, V), hidden_flat.reshape(B, S, D)


# speedup vs baseline: 3.2486x; 3.2486x over previous
import functools

import jax
import jax.numpy as jnp
from jax import lax
from jax.experimental import pallas as pl
from jax.experimental.pallas import tpu as pltpu
from jax.experimental.pallas import tpu_sc as plsc

B, S, D = 4, 8192, 8
V = 64
NW = 32
ST = S // 128
ST_PW = B * ST // NW
TPW = ST_PW * 128
IN_PW = TPW * D
OUT_PW = TPW * V
MAGIC = 12582912.0


@functools.partial(
    pl.kernel,
    mesh=plsc.VectorSubcoreMesh(core_axis_name="c", subcore_axis_name="s"),
    out_type=(
        jax.ShapeDtypeStruct((B * S * V,), jnp.float32),
        jax.ShapeDtypeStruct((B * S * D,), jnp.float32),
    ),
    compiler_params=pltpu.CompilerParams(needs_layout_passes=False),
    scratch_types=[
        pltpu.VMEM((IN_PW,), jnp.float32),
        pltpu.VMEM((OUT_PW,), jnp.float32),
        pltpu.SemaphoreType.DMA,
        pltpu.SemaphoreType.DMA,
    ],
)
def _sc_fake_logits(x_hbm, out_hbm, hid_hbm, x_v, o_v, in_sem, out_sem):
    cid = lax.axis_index("c")
    sid = lax.axis_index("s")
    wid = sid * 2 + cid
    b = lax.shift_right_logical(wid, 3)
    st0 = lax.bitwise_and(wid, 7) * ST_PW
    lane = lax.iota(jnp.int32, 16)

    in_off = b * (S * D) + st0 * (128 * D)
    in_cp = pltpu.async_copy(x_hbm.at[pl.ds(in_off, IN_PW)], x_v, in_sem)

    zeros = jnp.zeros((16,), jnp.float32)

    def zbody(i, carry):
        o_v[pl.ds(i * 16, 16)] = zeros
        return carry

    lax.fori_loop(0, OUT_PW // 16, zbody, 0, unroll=8)
    in_cp.wait()

    hid_cp = pltpu.async_copy(x_v, hid_hbm.at[pl.ds(in_off, IN_PW)], in_sem)

    def gbody(g, carry):
        base = lax.shift_right_logical(g, 3) * (128 * D) + lax.bitwise_and(g, 7) * 16
        xv = x_v[pl.ds(base, 16)]
        r = (xv + MAGIC) - MAGIC
        r = jnp.maximum(r, jnp.float32(0.0))
        idx = r.astype(jnp.int32) & (V - 1)
        val = idx.astype(jnp.float32) / jnp.float32(10.0)
        off = (
            jax.lax.shift_right_logical(idx, 3) * (128 * V)
            + jnp.bitwise_and(idx, 7) * 128
            + base
            + lane
        )
        plsc.store_scatter(o_v, [off], val)
        return carry

    lax.fori_loop(0, TPW // 16, gbody, 0, unroll=4)

    out_cps = []
    for vt in range(V // 8):
        dst = b * (S * V) + vt * (S * 8) + st0 * (128 * 8)
        out_cps.append(
            pltpu.async_copy(
                o_v.at[pl.ds(vt * (ST_PW * 128 * 8), ST_PW * 128 * 8)],
                out_hbm.at[pl.ds(dst, ST_PW * 128 * 8)],
                out_sem,
            )
        )
    for cp in out_cps:
        cp.wait()
    hid_cp.wait()


def kernel(inputs_embeds):
    x_phys = (
        inputs_embeds.reshape(B, ST, 128, D).transpose(0, 1, 3, 2).reshape(-1)
    )
    out_flat, hid_flat = _sc_fake_logits(x_phys)
    logits = (
        out_flat.reshape(B, V // 8, ST, 8, 128)
        .transpose(0, 2, 4, 1, 3)
        .reshape(B, S, V)
    )
    hidden = (
        hid_flat.reshape(B, ST, D, 128).transpose(0, 1, 3, 2).reshape(B, S, D)
    )
    return logits, hidden
